# baseline (speedup 1.0000x reference)
import jax
import jax.numpy as jnp
from jax import lax
from jax.experimental import pallas as pl
from jax.experimental.pallas import tpu as pltpu
from jax.experimental.pallas import tpu_sc as plsc

B = 262144
K = 256
V = 4
BN = 16384

_NC = 2
_NS = 16
_NW = _NC * _NS
_BPW = B // _NW
_L = 16


def _tc_body(xt_ref, tlut_ref, state_ref):
    x = xt_ref[...]
    tl = tlut_ref[...]
    t2 = jnp.sum(tl * tl, axis=1, keepdims=True)
    xt = lax.dot_general(tl, x, (((1,), (0,)), ((), ())),
                         preferred_element_type=jnp.float32)
    d2 = t2 - 2.0 * xt
    m = jnp.min(d2, axis=0, keepdims=True)
    sub = lax.broadcasted_iota(jnp.int32, (K, BN), 0)
    idx = jnp.min(jnp.where(d2 == m, sub, K), axis=0,
                  keepdims=True)
    state_ref[...] = idx[0]


def _sc_body(tlutT_hbm, state_hbm, out_hbm, tT_v, idx_v, out_v):
    wid = lax.axis_index("s") * _NC + lax.axis_index("c")
    base = wid * _BPW
    pltpu.sync_copy(tlutT_hbm, tT_v)
    pltpu.sync_copy(state_hbm.at[pl.ds(base, _BPW)], idx_v)

    @plsc.parallel_loop(0, _BPW // _L, unroll=16)
    def _step(i):
        s16 = idx_v[pl.ds(i * _L, _L)]
        off = (i // 8) * (V * 128) + (i % 8) * _L
        for j in range(V):
            out_v[pl.ds(off + j * 128, _L)] = plsc.load_gather(
                tT_v, [s16 + (j * K)])
    pltpu.sync_copy(out_v, out_hbm.at[pl.ds(base * V, _BPW * V)])


def kernel(X, tlut):
    state = pl.pallas_call(
        _tc_body,
        grid=(B // BN,),
        in_specs=[
            pl.BlockSpec((V, BN), lambda i: (0, i)),
            pl.BlockSpec((K, V), lambda i: (0, 0)),
        ],
        out_specs=pl.BlockSpec((BN,), lambda i: (i,)),
        out_shape=jax.ShapeDtypeStruct((B,), jnp.int32),
    )(X.T, tlut)

    hat_flat = pl.kernel(
        _sc_body,
        out_type=jax.ShapeDtypeStruct((V * B,), jnp.float32),
        mesh=plsc.VectorSubcoreMesh(core_axis_name="c", subcore_axis_name="s"),
        compiler_params=pltpu.CompilerParams(needs_layout_passes=False),
        scratch_types=[
            pltpu.VMEM((V * K,), jnp.float32),
            pltpu.VMEM((_BPW,), jnp.int32),
            pltpu.VMEM((V * _BPW,), jnp.float32),
        ],
    )(tlut.T.reshape(V * K), state)
    hatX = hat_flat.reshape(B // 128, V, 128).transpose(0, 2, 1).reshape(B, V)
    return hatX, state
